# Initial kernel scaffold; baseline (speedup 1.0000x reference)
import jax
import jax.numpy as jnp
from jax.experimental import pallas as pl


def kernel(img):
    raise NotImplementedError("write your pallas kernel here")


# speedup vs baseline: 75.5958x; 75.5958x over previous
import functools

import numpy as np
import jax
import jax.numpy as jnp
from jax import lax
from jax.experimental import pallas as pl
from jax.experimental.pallas import tpu as pltpu
from jax.experimental.pallas import tpu_sc as plsc

C, H, W = 3, 512, 512
GRID = 8
T = 64
NTILES = GRID * GRID
L = 16
NBINS = 256
CL = 64.0

_rng = np.random.RandomState(0)
_ALPHAS = np.array([_rng.uniform(0.5, 1.0) for _ in range(NTILES)],
                   dtype=np.float32)

_mesh = plsc.VectorSubcoreMesh(core_axis_name="c", subcore_axis_name="s")


@functools.partial(
    pl.kernel,
    mesh=_mesh,
    out_type=jax.ShapeDtypeStruct((C, H, W), jnp.float32),
    compiler_params=pltpu.CompilerParams(use_tc_tiling_on_sc=False,
                                          needs_layout_passes=False),
    scratch_types=[
        pltpu.VMEM((C, T, T), jnp.float32),
        pltpu.VMEM((T, T), jnp.float32),
        pltpu.VMEM((L, NBINS), jnp.float32),
        pltpu.VMEM((NBINS,), jnp.float32),
        pltpu.VMEM((NTILES,), jnp.float32),
    ],
)
def _clahe(img_hbm, alphas_hbm, out_hbm, chan_v, lum_v, hist_v, cdf_v,
           alpha_v):
    wid = lax.axis_index("s") * 2 + lax.axis_index("c")
    pltpu.sync_copy(alphas_hbm, alpha_v)

    lanes = lax.iota(jnp.int32, L)
    ones = jnp.ones((L,), jnp.float32)
    zeros = jnp.zeros((L,), jnp.float32)

    for tslot in range(2):
        t = wid + tslot * 32
        hs = (t // GRID) * T
        ws = (t % GRID) * T
        pltpu.sync_copy(img_hbm.at[:, pl.ds(hs, T), pl.ds(ws, T)], chan_v)

        def _zero(i, carry):
            for j in range(NBINS // L):
                hist_v[i, pl.ds(j * L, L)] = zeros
            return carry
        lax.fori_loop(0, L, _zero, 0)

        def _pass_a(r, mm):
            vmin, vmax = mm
            for c4 in range(T // L):
                sl = pl.ds(c4 * L, L)
                lum = (chan_v[0, r, sl] + chan_v[1, r, sl]
                       + chan_v[2, r, sl]) / 3.0
                lum_v[r, sl] = lum
                vmin = jnp.minimum(vmin, lum)
                vmax = jnp.maximum(vmax, lum)
                bins = jnp.clip((lum * 256.0).astype(jnp.int32), 0, 255)
                plsc.addupdate_scatter(hist_v, [lanes, bins], ones)
            return vmin, vmax
        vmin, vmax = lax.fori_loop(
            0, T, _pass_a,
            (jnp.full((L,), jnp.inf, jnp.float32),
             jnp.full((L,), -jnp.inf, jnp.float32)))
        tmin_v = jnp.broadcast_to(jnp.min(vmin), (L,))
        tmax_v = jnp.broadcast_to(jnp.max(vmax), (L,))

        exc = zeros
        for j in range(NBINS // L):
            sl = pl.ds(j * L, L)
            def _sum_rows(i, acc, sl=sl):
                return acc + hist_v[i, sl]
            hj = lax.fori_loop(0, L, _sum_rows, zeros)
            exc = exc + jnp.maximum(hj - CL, 0.0)
            cdf_v[sl] = jnp.minimum(hj, CL)
        add_v = jnp.broadcast_to(jnp.sum(exc), (L,)) * (1.0 / 256.0)

        run = zeros
        for j in range(NBINS // L):
            sl = pl.ds(j * L, L)
            v = cdf_v[sl] + add_v
            cdf_v[sl] = plsc.cumsum(v) + run
            run = run + jnp.broadcast_to(jnp.sum(v), (L,))
        total_v = run

        a_v = plsc.load_gather(alpha_v, [jnp.broadcast_to(t, (L,))])
        hr_v = tmax_v > tmin_v
        denom_v = jnp.where(hr_v, tmax_v - tmin_v, 1.0)

        def _pass_b(r, carry):
            for c4 in range(T // L):
                sl = pl.ds(c4 * L, L)
                lum = lum_v[r, sl]
                norm = (lum - tmin_v) / denom_v
                idx = jnp.clip((norm * 255.0).astype(jnp.int32), 0, 255)
                enh = plsc.load_gather(cdf_v, [idx]) / total_v
                et = a_v * enh + (1.0 - a_v) * lum
                mask = lum > 1e-05
                safe = jnp.where(mask, lum, 1.0)
                for ch in range(C):
                    orig = chan_v[ch, r, sl]
                    ratio = jnp.where(mask, orig / safe, 1.0)
                    val = jnp.where(hr_v, et * ratio, orig)
                    chan_v[ch, r, sl] = jnp.clip(val, 0.0, 1.0)
            return carry
        lax.fori_loop(0, T, _pass_b, 0)

        pltpu.sync_copy(chan_v, out_hbm.at[:, pl.ds(hs, T), pl.ds(ws, T)])


def kernel(img):
    return _clahe(img, jnp.asarray(_ALPHAS))


# speedup vs baseline: 76.0734x; 1.0063x over previous
import functools

import numpy as np
import jax
import jax.numpy as jnp
from jax import lax
from jax.experimental import pallas as pl
from jax.experimental.pallas import tpu as pltpu
from jax.experimental.pallas import tpu_sc as plsc

C, H, W = 3, 512, 512
GRID = 8
T = 64
NTILES = GRID * GRID
L = 16
NBINS = 256
NCH = NBINS // L
CL = 64.0

_rng = np.random.RandomState(0)
_ALPHAS = np.array([_rng.uniform(0.5, 1.0) for _ in range(NTILES)],
                   dtype=np.float32)

_mesh = plsc.VectorSubcoreMesh(core_axis_name="c", subcore_axis_name="s")


@functools.partial(
    pl.kernel,
    mesh=_mesh,
    out_type=jax.ShapeDtypeStruct((C, H, W), jnp.float32),
    compiler_params=pltpu.CompilerParams(use_tc_tiling_on_sc=False,
                                         needs_layout_passes=False),
    scratch_types=[
        pltpu.VMEM((C, T, T), jnp.float32),
        pltpu.VMEM((T, T), jnp.float32),
        pltpu.VMEM((L, NBINS), jnp.float32),
        pltpu.VMEM((NBINS,), jnp.float32),
        pltpu.VMEM((NTILES,), jnp.float32),
    ],
)
def _clahe(img_hbm, alphas_hbm, out_hbm, chan_v, lum_v, hist_v, cdf_v,
           alpha_v):
    wid = lax.axis_index("s") * 2 + lax.axis_index("c")
    pltpu.sync_copy(alphas_hbm, alpha_v)

    lanes = lax.iota(jnp.int32, L)
    ones = jnp.ones((L,), jnp.float32)
    zeros = jnp.zeros((L,), jnp.float32)

    def _zero(i, carry):
        for j in range(NCH):
            hist_v[i, pl.ds(j * L, L)] = zeros
        return carry
    lax.fori_loop(0, L, _zero, 0)

    for tslot in range(2):
        t = wid + tslot * 32
        hs = (t // GRID) * T
        ws = (t % GRID) * T
        pltpu.sync_copy(img_hbm.at[:, pl.ds(hs, T), pl.ds(ws, T)], chan_v)

        def _pass_a(r, mm):
            vmin, vmax = mm
            for c4 in range(T // L):
                sl = pl.ds(c4 * L, L)
                lum = (chan_v[0, r, sl] + chan_v[1, r, sl]
                       + chan_v[2, r, sl]) * (1.0 / 3.0)
                lum_v[r, sl] = lum
                vmin = jnp.minimum(vmin, lum)
                vmax = jnp.maximum(vmax, lum)
                bins = jnp.clip((lum * 256.0).astype(jnp.int32), 0, 255)
                plsc.addupdate_scatter(hist_v, [lanes, bins], ones)
            return vmin, vmax
        vmin, vmax = lax.fori_loop(
            0, T, _pass_a,
            (jnp.full((L,), jnp.inf, jnp.float32),
             jnp.full((L,), -jnp.inf, jnp.float32)))
        tmin_v = jnp.broadcast_to(jnp.min(vmin), (L,))
        tmax_v = jnp.broadcast_to(jnp.max(vmax), (L,))

        def _sum_rows(i, accs):
            out = []
            for j in range(NCH):
                sl = pl.ds(j * L, L)
                out.append(accs[j] + hist_v[i, sl])
                hist_v[i, sl] = zeros
            return tuple(out)
        accs = lax.fori_loop(0, L, _sum_rows, (zeros,) * NCH)
        exc = zeros
        clipped = []
        for j in range(NCH):
            hj = accs[j]
            exc = exc + jnp.maximum(hj - CL, 0.0)
            clipped.append(jnp.minimum(hj, CL))
        add_v = jnp.broadcast_to(jnp.sum(exc), (L,)) * (1.0 / 256.0)

        run = zeros
        for j in range(NCH):
            sl = pl.ds(j * L, L)
            cdf_v[sl] = plsc.cumsum(clipped[j] + add_v) + run
            run = plsc.load_gather(cdf_v, [jnp.full((L,), j * L + L - 1,
                                                    jnp.int32)])
        inv_total = 1.0 / run
        for j in range(NCH):
            sl = pl.ds(j * L, L)
            cdf_v[sl] = cdf_v[sl] * inv_total

        a_v = plsc.load_gather(alpha_v, [jnp.broadcast_to(t, (L,))])
        hr_v = tmax_v > tmin_v
        scale_v = jnp.where(hr_v, 255.0 / (tmax_v - tmin_v), 0.0)

        def _pass_b(r, carry):
            for c4 in range(T // L):
                sl = pl.ds(c4 * L, L)
                lum = lum_v[r, sl]
                idx = jnp.clip(((lum - tmin_v) * scale_v).astype(jnp.int32),
                               0, 255)
                enh = plsc.load_gather(cdf_v, [idx])
                et = lum + a_v * (enh - lum)
                mask = lum > 1e-05
                g = et * (1.0 / jnp.where(mask, lum, 1.0))
                for ch in range(C):
                    orig = chan_v[ch, r, sl]
                    val = jnp.where(mask, g * orig, et)
                    val = jnp.where(hr_v, val, orig)
                    chan_v[ch, r, sl] = jnp.clip(val, 0.0, 1.0)
            return carry
        lax.fori_loop(0, T, _pass_b, 0)

        pltpu.sync_copy(chan_v, out_hbm.at[:, pl.ds(hs, T), pl.ds(ws, T)])


def kernel(img):
    return _clahe(img, jnp.asarray(_ALPHAS))


# speedup vs baseline: 84.7811x; 1.1145x over previous
import functools

import numpy as np
import jax
import jax.numpy as jnp
from jax import lax
from jax.experimental import pallas as pl
from jax.experimental.pallas import tpu as pltpu
from jax.experimental.pallas import tpu_sc as plsc

C, H, W = 3, 512, 512
GRID = 8
T = 64
NTILES = GRID * GRID
L = 16
NBINS = 256
NCH = NBINS // L
CL = 64.0

_rng = np.random.RandomState(0)
_ALPHAS = np.array([_rng.uniform(0.5, 1.0) for _ in range(NTILES)],
                   dtype=np.float32)

_mesh = plsc.VectorSubcoreMesh(core_axis_name="c", subcore_axis_name="s")


@functools.partial(
    pl.kernel,
    mesh=_mesh,
    out_type=jax.ShapeDtypeStruct((C, H, W), jnp.float32),
    compiler_params=pltpu.CompilerParams(use_tc_tiling_on_sc=True,
                                         needs_layout_passes=False),
    scratch_types=[
        pltpu.VMEM((C, T, 2 * T), jnp.float32),
        pltpu.VMEM((T, 2 * T), jnp.float32),
        pltpu.VMEM((L, NBINS), jnp.float32),
        pltpu.VMEM((NBINS,), jnp.float32),
        pltpu.VMEM((NTILES,), jnp.float32),
    ],
)
def _clahe(img_hbm, alphas_hbm, out_hbm, chan_v, lum_v, hist_v, cdf_v,
           alpha_v):
    wid = lax.axis_index("s") * 2 + lax.axis_index("c")
    pltpu.sync_copy(alphas_hbm, alpha_v)

    lanes = lax.iota(jnp.int32, L)
    ones = jnp.ones((L,), jnp.float32)
    zeros = jnp.zeros((L,), jnp.float32)

    def _zero(i, carry):
        for j in range(NCH):
            hist_v[i, pl.ds(j * L, L)] = zeros
        return carry
    lax.fori_loop(0, L, _zero, 0)

    iB = wid // 4
    jB = wid % 4
    hs = iB * T
    ws = jB * (2 * T)
    pltpu.sync_copy(img_hbm.at[:, pl.ds(hs, T), pl.ds(ws, 2 * T)], chan_v)

    for half in range(2):
        t = iB * GRID + jB * 2 + half
        wo = half * T

        def _pass_a(r, mm):
            vmin, vmax = mm
            for c4 in range(T // L):
                sl = pl.ds(wo + c4 * L, L)
                lum = (chan_v[0, r, sl] + chan_v[1, r, sl]
                       + chan_v[2, r, sl]) * (1.0 / 3.0)
                lum_v[r, sl] = lum
                vmin = jnp.minimum(vmin, lum)
                vmax = jnp.maximum(vmax, lum)
                bins = jnp.clip((lum * 256.0).astype(jnp.int32), 0, 255)
                plsc.addupdate_scatter(hist_v, [lanes, bins], ones)
            return vmin, vmax
        vmin, vmax = lax.fori_loop(
            0, T, _pass_a,
            (jnp.full((L,), jnp.inf, jnp.float32),
             jnp.full((L,), -jnp.inf, jnp.float32)))
        tmin_v = jnp.broadcast_to(jnp.min(vmin), (L,))
        tmax_v = jnp.broadcast_to(jnp.max(vmax), (L,))

        def _sum_rows(i, accs):
            out = []
            for j in range(NCH):
                sl = pl.ds(j * L, L)
                out.append(accs[j] + hist_v[i, sl])
                hist_v[i, sl] = zeros
            return tuple(out)
        accs = lax.fori_loop(0, L, _sum_rows, (zeros,) * NCH)
        exc = zeros
        clipped = []
        for j in range(NCH):
            hj = accs[j]
            exc = exc + jnp.maximum(hj - CL, 0.0)
            clipped.append(jnp.minimum(hj, CL))
        add_v = jnp.broadcast_to(jnp.sum(exc), (L,)) * (1.0 / 256.0)

        run = zeros
        for j in range(NCH):
            sl = pl.ds(j * L, L)
            cdf_v[sl] = plsc.cumsum(clipped[j] + add_v) + run
            run = plsc.load_gather(cdf_v, [jnp.full((L,), j * L + L - 1,
                                                    jnp.int32)])
        inv_total = 1.0 / run
        for j in range(NCH):
            sl = pl.ds(j * L, L)
            cdf_v[sl] = cdf_v[sl] * inv_total

        a_v = plsc.load_gather(alpha_v, [jnp.broadcast_to(t, (L,))])
        hr_v = tmax_v > tmin_v
        scale_v = jnp.where(hr_v, 255.0 / (tmax_v - tmin_v), 0.0)

        def _pass_b(r, carry):
            for c4 in range(T // L):
                sl = pl.ds(wo + c4 * L, L)
                lum = lum_v[r, sl]
                idx = jnp.clip(((lum - tmin_v) * scale_v).astype(jnp.int32),
                               0, 255)
                enh = plsc.load_gather(cdf_v, [idx])
                et = lum + a_v * (enh - lum)
                mask = lum > 1e-05
                g = et * (1.0 / jnp.where(mask, lum, 1.0))
                for ch in range(C):
                    orig = chan_v[ch, r, sl]
                    val = jnp.where(mask, g * orig, et)
                    val = jnp.where(hr_v, val, orig)
                    chan_v[ch, r, sl] = jnp.clip(val, 0.0, 1.0)
            return carry
        lax.fori_loop(0, T, _pass_b, 0)

    pltpu.sync_copy(chan_v, out_hbm.at[:, pl.ds(hs, T), pl.ds(ws, 2 * T)])


def kernel(img):
    return _clahe(img, jnp.asarray(_ALPHAS))


# speedup vs baseline: 109.0207x; 1.2859x over previous
import functools

import numpy as np
import jax
import jax.numpy as jnp
from jax import lax
from jax.experimental import pallas as pl
from jax.experimental.pallas import tpu as pltpu
from jax.experimental.pallas import tpu_sc as plsc

C, H, W = 3, 512, 512
GRID = 8
T = 64
NTILES = GRID * GRID
L = 16
NBINS = 256
NCH = NBINS // L
NCK = T // L
CL = 64.0

_rng = np.random.RandomState(0)
_ALPHAS = np.array([_rng.uniform(0.5, 1.0) for _ in range(NTILES)],
                   dtype=np.float32)

_mesh = plsc.VectorSubcoreMesh(core_axis_name="c", subcore_axis_name="s")


@functools.partial(
    pl.kernel,
    mesh=_mesh,
    out_type=jax.ShapeDtypeStruct((C, H, W), jnp.float32),
    compiler_params=pltpu.CompilerParams(use_tc_tiling_on_sc=True,
                                         needs_layout_passes=False),
    scratch_types=[
        pltpu.VMEM((C, T, 2 * T), jnp.float32),
        pltpu.VMEM((L * NBINS,), jnp.float32),
        pltpu.VMEM((NBINS,), jnp.float32),
        pltpu.VMEM((NTILES,), jnp.float32),
    ],
)
def _clahe(img_hbm, alphas_hbm, out_hbm, chan_v, hist_v, cdf_v, alpha_v):
    wid = lax.axis_index("s") * 2 + lax.axis_index("c")
    pltpu.sync_copy(alphas_hbm, alpha_v)

    lanes256 = lax.iota(jnp.int32, L) * NBINS
    ones = jnp.ones((L,), jnp.float32)
    zeros = jnp.zeros((L,), jnp.float32)

    def _zero(i, carry):
        hist_v[pl.ds(i * L, L)] = zeros
        return carry
    lax.fori_loop(0, L * NCH, _zero, 0)

    iB = wid // 4
    jB = wid % 4
    hs = iB * T
    ws = jB * (2 * T)
    pltpu.sync_copy(img_hbm.at[:, pl.ds(hs, T), pl.ds(ws, 2 * T)], chan_v)

    for half in range(2):
        t = iB * GRID + jB * 2 + half
        wo = half * T

        def _pass_a(r, mm):
            vmin, vmax = mm
            px = [[chan_v[ch, r, pl.ds(wo + k * L, L)] for ch in range(C)]
                  for k in range(NCK)]
            idxs = []
            for k in range(NCK):
                lum = (px[k][0] + px[k][1] + px[k][2]) * (1.0 / 3.0)
                vmin = jnp.minimum(vmin, lum)
                vmax = jnp.maximum(vmax, lum)
                bins = jnp.clip((lum * 256.0).astype(jnp.int32), 0, 255)
                idxs.append(lanes256 + bins)
            for k in range(NCK):
                plsc.addupdate_scatter(hist_v, [idxs[k]], ones)
            return vmin, vmax
        vmin, vmax = lax.fori_loop(
            0, T, _pass_a,
            (jnp.full((L,), jnp.inf, jnp.float32),
             jnp.full((L,), -jnp.inf, jnp.float32)))
        tmin_v = jnp.broadcast_to(jnp.min(vmin), (L,))
        tmax_v = jnp.broadcast_to(jnp.max(vmax), (L,))

        def _sum_rows(i, accs):
            loads = [hist_v[pl.ds(i * NBINS + j * L, L)] for j in range(NCH)]
            for j in range(NCH):
                hist_v[pl.ds(i * NBINS + j * L, L)] = zeros
            return tuple(accs[j] + loads[j] for j in range(NCH))
        accs = lax.fori_loop(0, L, _sum_rows, (zeros,) * NCH)
        exc = zeros
        clipped = []
        for j in range(NCH):
            hj = accs[j]
            exc = exc + jnp.maximum(hj - CL, 0.0)
            clipped.append(jnp.minimum(hj, CL))
        add_v = jnp.broadcast_to(jnp.sum(exc), (L,)) * (1.0 / 256.0)

        run = zeros
        for j in range(NCH):
            sl = pl.ds(j * L, L)
            cdf_v[sl] = plsc.cumsum(clipped[j] + add_v) + run
            run = plsc.load_gather(cdf_v, [jnp.full((L,), j * L + L - 1,
                                                    jnp.int32)])
        inv_total = 1.0 / run
        for j in range(NCH):
            sl = pl.ds(j * L, L)
            cdf_v[sl] = cdf_v[sl] * inv_total

        a_v = plsc.load_gather(alpha_v, [jnp.broadcast_to(t, (L,))])
        hr_v = tmax_v > tmin_v
        scale_v = jnp.where(hr_v, 255.0 / (tmax_v - tmin_v), 0.0)

        def _pass_b(r, carry):
            px = [[chan_v[ch, r, pl.ds(wo + k * L, L)] for ch in range(C)]
                  for k in range(NCK)]
            outs = []
            for k in range(NCK):
                lum = (px[k][0] + px[k][1] + px[k][2]) * (1.0 / 3.0)
                idx = jnp.clip(((lum - tmin_v) * scale_v).astype(jnp.int32),
                               0, 255)
                enh = plsc.load_gather(cdf_v, [idx])
                et = lum + a_v * (enh - lum)
                mask = lum > 1e-05
                g = et * (1.0 / jnp.where(mask, lum, 1.0))
                for ch in range(C):
                    orig = px[k][ch]
                    val = jnp.where(mask, g * orig, et)
                    val = jnp.where(hr_v, val, orig)
                    outs.append(jnp.clip(val, 0.0, 1.0))
            for k in range(NCK):
                for ch in range(C):
                    chan_v[ch, r, pl.ds(wo + k * L, L)] = outs[k * C + ch]
            return carry
        lax.fori_loop(0, T, _pass_b, 0)

    pltpu.sync_copy(chan_v, out_hbm.at[:, pl.ds(hs, T), pl.ds(ws, 2 * T)])


def kernel(img):
    return _clahe(img, jnp.asarray(_ALPHAS))


# speedup vs baseline: 111.6399x; 1.0240x over previous
import functools

import numpy as np
import jax
import jax.numpy as jnp
from jax import lax
from jax.experimental import pallas as pl
from jax.experimental.pallas import tpu as pltpu
from jax.experimental.pallas import tpu_sc as plsc

C, H, W = 3, 512, 512
GRID = 8
T = 64
NTILES = GRID * GRID
L = 16
NBINS = 256
NCH = NBINS // L
NCK = T // L
CL = 64.0

_rng = np.random.RandomState(0)
_ALPHAS = np.array([_rng.uniform(0.5, 1.0) for _ in range(NTILES)],
                   dtype=np.float32)

_mesh = plsc.VectorSubcoreMesh(core_axis_name="c", subcore_axis_name="s")


@functools.partial(
    pl.kernel,
    mesh=_mesh,
    out_type=jax.ShapeDtypeStruct((C, H, W), jnp.float32),
    compiler_params=pltpu.CompilerParams(use_tc_tiling_on_sc=True,
                                         needs_layout_passes=False),
    scratch_types=[
        pltpu.VMEM((C, T, 2 * T), jnp.float32),
        pltpu.VMEM((L * NBINS,), jnp.float32),
        pltpu.VMEM((NBINS,), jnp.float32),
        pltpu.VMEM((NTILES,), jnp.float32),
    ],
)
def _clahe(img_hbm, alphas_hbm, out_hbm, chan_v, hist_v, cdf_v, alpha_v):
    wid = lax.axis_index("s") * 2 + lax.axis_index("c")
    pltpu.sync_copy(alphas_hbm, alpha_v)

    lanes256 = lax.iota(jnp.int32, L) * NBINS
    ones = jnp.ones((L,), jnp.float32)
    zeros = jnp.zeros((L,), jnp.float32)

    def _zero(i, carry):
        for j in range(NCH):
            hist_v[pl.ds(i * NBINS + j * L, L)] = zeros
        return carry
    lax.fori_loop(0, L, _zero, 0)

    iB = wid // 4
    jB = wid % 4
    hs = iB * T
    ws = jB * (2 * T)
    pltpu.sync_copy(img_hbm.at[:, pl.ds(hs, T), pl.ds(ws, 2 * T)], chan_v)

    for half in range(2):
        t = iB * GRID + jB * 2 + half
        wo = half * T

        def _pass_a(r, mm):
            vmin, vmax = mm
            px = [[chan_v[ch, r, pl.ds(wo + k * L, L)] for ch in range(C)]
                  for k in range(NCK)]
            idxs = []
            for k in range(NCK):
                lum = (px[k][0] + px[k][1] + px[k][2]) * (1.0 / 3.0)
                vmin = jnp.minimum(vmin, lum)
                vmax = jnp.maximum(vmax, lum)
                bins = jnp.clip((lum * 256.0).astype(jnp.int32), 0, 255)
                idxs.append(lanes256 + bins)
            for k in range(NCK):
                plsc.addupdate_scatter(hist_v, [idxs[k]], ones)
            return vmin, vmax
        vmin, vmax = lax.fori_loop(
            0, T, _pass_a,
            (jnp.full((L,), jnp.inf, jnp.float32),
             jnp.full((L,), -jnp.inf, jnp.float32)))
        tmin_v = jnp.broadcast_to(jnp.min(vmin), (L,))
        tmax_v = jnp.broadcast_to(jnp.max(vmax), (L,))

        def _sum_rows(i, accs):
            loads = [hist_v[pl.ds(i * NBINS + j * L, L)] for j in range(NCH)]
            for j in range(NCH):
                hist_v[pl.ds(i * NBINS + j * L, L)] = zeros
            return tuple(accs[j] + loads[j] for j in range(NCH))
        accs = lax.fori_loop(0, L, _sum_rows, (zeros,) * NCH)
        exc = zeros
        clipped = []
        for j in range(NCH):
            hj = accs[j]
            exc = exc + jnp.maximum(hj - CL, 0.0)
            clipped.append(jnp.minimum(hj, CL))
        add_v = jnp.broadcast_to(jnp.sum(exc), (L,)) * (1.0 / 256.0)

        run = zeros
        for j in range(NCH):
            sl = pl.ds(j * L, L)
            cdf_v[sl] = plsc.cumsum(clipped[j] + add_v) + run
            run = plsc.load_gather(cdf_v, [jnp.full((L,), j * L + L - 1,
                                                    jnp.int32)])
        inv_total = 1.0 / run
        for j in range(NCH):
            sl = pl.ds(j * L, L)
            cdf_v[sl] = cdf_v[sl] * inv_total

        a_v = plsc.load_gather(alpha_v, [jnp.broadcast_to(t, (L,))])
        hr_v = tmax_v > tmin_v
        scale_v = jnp.where(hr_v, 255.0 / (tmax_v - tmin_v), 0.0)

        def _pass_b(r, carry):
            px = [[chan_v[ch, r, pl.ds(wo + k * L, L)] for ch in range(C)]
                  for k in range(NCK)]
            outs = []
            for k in range(NCK):
                lum = (px[k][0] + px[k][1] + px[k][2]) * (1.0 / 3.0)
                idx = jnp.clip(((lum - tmin_v) * scale_v).astype(jnp.int32),
                               0, 255)
                enh = plsc.load_gather(cdf_v, [idx])
                et = lum + a_v * (enh - lum)
                mask = lum > 1e-05
                g = et * (1.0 / jnp.where(mask, lum, 1.0))
                for ch in range(C):
                    orig = px[k][ch]
                    val = jnp.where(mask, g * orig, et)
                    val = jnp.where(hr_v, val, orig)
                    outs.append(jnp.clip(val, 0.0, 1.0))
            for k in range(NCK):
                for ch in range(C):
                    chan_v[ch, r, pl.ds(wo + k * L, L)] = outs[k * C + ch]
            return carry
        lax.fori_loop(0, T, _pass_b, 0)

    pltpu.sync_copy(chan_v, out_hbm.at[:, pl.ds(hs, T), pl.ds(ws, 2 * T)])


def kernel(img):
    return _clahe(img, jnp.asarray(_ALPHAS))


# speedup vs baseline: 116.3316x; 1.0420x over previous
import functools

import numpy as np
import jax
import jax.numpy as jnp
from jax import lax
from jax.experimental import pallas as pl
from jax.experimental.pallas import tpu as pltpu
from jax.experimental.pallas import tpu_sc as plsc

C, H, W = 3, 512, 512
GRID = 8
T = 64
NTILES = GRID * GRID
L = 16
NBINS = 256
NCH = NBINS // L
NCK = T // L
CL = 64.0

_rng = np.random.RandomState(0)
_ALPHAS = np.array([_rng.uniform(0.5, 1.0) for _ in range(NTILES)],
                   dtype=np.float32)

_mesh = plsc.VectorSubcoreMesh(core_axis_name="c", subcore_axis_name="s")


@functools.partial(
    pl.kernel,
    mesh=_mesh,
    out_type=jax.ShapeDtypeStruct((C, H, W), jnp.float32),
    compiler_params=pltpu.CompilerParams(use_tc_tiling_on_sc=True,
                                         needs_layout_passes=False),
    scratch_types=[
        pltpu.VMEM((C, T, 2 * T), jnp.float32),
        pltpu.VMEM((L * NBINS,), jnp.float32),
        pltpu.VMEM((NBINS,), jnp.float32),
        pltpu.VMEM((NTILES,), jnp.float32),
    ],
)
def _clahe(img_hbm, alphas_hbm, out_hbm, chan_v, hist_v, cdf_v, alpha_v):
    wid = lax.axis_index("s") * 2 + lax.axis_index("c")
    pltpu.sync_copy(alphas_hbm, alpha_v)

    lanes256 = lax.iota(jnp.int32, L) * NBINS
    ones = jnp.ones((L,), jnp.float32)
    zeros = jnp.zeros((L,), jnp.float32)

    def _zero(i, carry):
        for j in range(NCH):
            hist_v[pl.ds(i * NBINS + j * L, L)] = zeros
        return carry
    lax.fori_loop(0, L, _zero, 0)

    iB = wid // 4
    jB = wid % 4
    hs = iB * T
    ws = jB * (2 * T)
    pltpu.sync_copy(img_hbm.at[:, pl.ds(hs, T), pl.ds(ws, 2 * T)], chan_v)

    for half in range(2):
        t = iB * GRID + jB * 2 + half
        wo = half * T

        @plsc.parallel_loop(
            0, T, 1, unroll=2,
            carry=(jnp.full((L,), jnp.inf, jnp.float32),
                   jnp.full((L,), -jnp.inf, jnp.float32)))
        def _pass_a(r, mm):
            vmin, vmax = mm
            px = [[chan_v[ch, r, pl.ds(wo + k * L, L)] for ch in range(C)]
                  for k in range(NCK)]
            idxs = []
            for k in range(NCK):
                lum = (px[k][0] + px[k][1] + px[k][2]) * (1.0 / 3.0)
                vmin = jnp.minimum(vmin, lum)
                vmax = jnp.maximum(vmax, lum)
                bins = jnp.clip((lum * 256.0).astype(jnp.int32), 0, 255)
                idxs.append(lanes256 + bins)
            for k in range(NCK):
                plsc.addupdate_scatter(hist_v, [idxs[k]], ones)
            return vmin, vmax
        vmin, vmax = _pass_a
        tmin_v = jnp.broadcast_to(jnp.min(vmin), (L,))
        tmax_v = jnp.broadcast_to(jnp.max(vmax), (L,))

        @plsc.parallel_loop(0, L, 1, unroll=2, carry=(zeros,) * NCH)
        def _sum_rows(i, accs):
            loads = [hist_v[pl.ds(i * NBINS + j * L, L)] for j in range(NCH)]
            for j in range(NCH):
                hist_v[pl.ds(i * NBINS + j * L, L)] = zeros
            return tuple(accs[j] + loads[j] for j in range(NCH))
        accs = _sum_rows
        exc = zeros
        clipped = []
        for j in range(NCH):
            hj = accs[j]
            exc = exc + jnp.maximum(hj - CL, 0.0)
            clipped.append(jnp.minimum(hj, CL))
        add_v = jnp.broadcast_to(jnp.sum(exc), (L,)) * (1.0 / 256.0)

        run = zeros
        for j in range(NCH):
            sl = pl.ds(j * L, L)
            cdf_v[sl] = plsc.cumsum(clipped[j] + add_v) + run
            run = plsc.load_gather(cdf_v, [jnp.full((L,), j * L + L - 1,
                                                    jnp.int32)])
        inv_total = 1.0 / run
        for j in range(NCH):
            sl = pl.ds(j * L, L)
            cdf_v[sl] = cdf_v[sl] * inv_total

        a_v = plsc.load_gather(alpha_v, [jnp.broadcast_to(t, (L,))])
        hr_v = tmax_v > tmin_v
        scale_v = jnp.where(hr_v, 255.0 / (tmax_v - tmin_v), 0.0)

        @plsc.parallel_loop(0, T, 1, unroll=2)
        def _pass_b(r):
            px = [[chan_v[ch, r, pl.ds(wo + k * L, L)] for ch in range(C)]
                  for k in range(NCK)]
            outs = []
            for k in range(NCK):
                lum = (px[k][0] + px[k][1] + px[k][2]) * (1.0 / 3.0)
                idx = jnp.clip(((lum - tmin_v) * scale_v).astype(jnp.int32),
                               0, 255)
                enh = plsc.load_gather(cdf_v, [idx])
                et = lum + a_v * (enh - lum)
                mask = lum > 1e-05
                g = et * (1.0 / jnp.where(mask, lum, 1.0))
                for ch in range(C):
                    orig = px[k][ch]
                    val = jnp.where(mask, g * orig, et)
                    val = jnp.where(hr_v, val, orig)
                    outs.append(jnp.clip(val, 0.0, 1.0))
            for k in range(NCK):
                for ch in range(C):
                    chan_v[ch, r, pl.ds(wo + k * L, L)] = outs[k * C + ch]

    pltpu.sync_copy(chan_v, out_hbm.at[:, pl.ds(hs, T), pl.ds(ws, 2 * T)])


def kernel(img):
    return _clahe(img, jnp.asarray(_ALPHAS))


# speedup vs baseline: 121.1978x; 1.0418x over previous
import functools

import numpy as np
import jax
import jax.numpy as jnp
from jax import lax
from jax.experimental import pallas as pl
from jax.experimental.pallas import tpu as pltpu
from jax.experimental.pallas import tpu_sc as plsc

C, H, W = 3, 512, 512
GRID = 8
T = 64
NTILES = GRID * GRID
L = 16
NBINS = 256
NCH = NBINS // L
NCK = T // L
CL = 64.0

_rng = np.random.RandomState(0)
_ALPHAS = np.array([_rng.uniform(0.5, 1.0) for _ in range(NTILES)],
                   dtype=np.float32)

_mesh = plsc.VectorSubcoreMesh(core_axis_name="c", subcore_axis_name="s")


@functools.partial(
    pl.kernel,
    mesh=_mesh,
    out_type=jax.ShapeDtypeStruct((C, H, W), jnp.float32),
    compiler_params=pltpu.CompilerParams(use_tc_tiling_on_sc=True,
                                         needs_layout_passes=False),
    scratch_types=[
        pltpu.VMEM((C, T, 2 * T), jnp.float32),
        pltpu.VMEM((L * NBINS,), jnp.float32),
        pltpu.VMEM((NBINS,), jnp.float32),
        pltpu.VMEM((NTILES,), jnp.float32),
    ],
)
def _clahe(img_hbm, alphas_hbm, out_hbm, chan_v, hist_v, cdf_v, alpha_v):
    wid = lax.axis_index("s") * 2 + lax.axis_index("c")
    pltpu.sync_copy(alphas_hbm, alpha_v)

    lanes256 = lax.iota(jnp.int32, L) * NBINS
    ones = jnp.ones((L,), jnp.float32)
    zeros = jnp.zeros((L,), jnp.float32)

    def _zero(i, carry):
        for j in range(NCH):
            hist_v[pl.ds(i * NBINS + j * L, L)] = zeros
        return carry
    lax.fori_loop(0, L, _zero, 0)

    iB = wid // 4
    jB = wid % 4
    hs = iB * T
    ws = jB * (2 * T)
    pltpu.sync_copy(img_hbm.at[:, pl.ds(hs, T), pl.ds(ws, 2 * T)], chan_v)

    def _do_half(half, carry):
        t = iB * GRID + jB * 2 + half
        wo = half * T

        @plsc.parallel_loop(
            0, T, 1, unroll=2,
            carry=(jnp.full((L,), jnp.inf, jnp.float32),
                   jnp.full((L,), -jnp.inf, jnp.float32)))
        def _pass_a(r, mm):
            vmin, vmax = mm
            px = [[chan_v[ch, r, pl.ds(wo + k * L, L)] for ch in range(C)]
                  for k in range(NCK)]
            idxs = []
            for k in range(NCK):
                lum = (px[k][0] + px[k][1] + px[k][2]) * (1.0 / 3.0)
                vmin = jnp.minimum(vmin, lum)
                vmax = jnp.maximum(vmax, lum)
                bins = jnp.clip((lum * 256.0).astype(jnp.int32), 0, 255)
                idxs.append(lanes256 + bins)
            for k in range(NCK):
                plsc.addupdate_scatter(hist_v, [idxs[k]], ones)
            return vmin, vmax
        vmin, vmax = _pass_a
        tmin_v = jnp.broadcast_to(jnp.min(vmin), (L,))
        tmax_v = jnp.broadcast_to(jnp.max(vmax), (L,))

        @plsc.parallel_loop(0, L, 1, unroll=2, carry=(zeros,) * NCH)
        def _sum_rows(i, accs):
            loads = [hist_v[pl.ds(i * NBINS + j * L, L)] for j in range(NCH)]
            for j in range(NCH):
                hist_v[pl.ds(i * NBINS + j * L, L)] = zeros
            return tuple(accs[j] + loads[j] for j in range(NCH))
        accs = _sum_rows
        exc = zeros
        clipped = []
        for j in range(NCH):
            hj = accs[j]
            exc = exc + jnp.maximum(hj - CL, 0.0)
            clipped.append(jnp.minimum(hj, CL))
        add_v = jnp.broadcast_to(jnp.sum(exc), (L,)) * (1.0 / 256.0)

        run = zeros
        for j in range(NCH):
            sl = pl.ds(j * L, L)
            cdf_v[sl] = plsc.cumsum(clipped[j] + add_v) + run
            run = plsc.load_gather(cdf_v, [jnp.full((L,), j * L + L - 1,
                                                    jnp.int32)])
        inv_total = 1.0 / run
        for j in range(NCH):
            sl = pl.ds(j * L, L)
            cdf_v[sl] = cdf_v[sl] * inv_total

        a_v = plsc.load_gather(alpha_v, [jnp.broadcast_to(t, (L,))])
        hr_v = tmax_v > tmin_v
        scale_v = jnp.where(hr_v, 255.0 / (tmax_v - tmin_v), 0.0)

        @plsc.parallel_loop(0, T, 1, unroll=2)
        def _pass_b(r):
            px = [[chan_v[ch, r, pl.ds(wo + k * L, L)] for ch in range(C)]
                  for k in range(NCK)]
            outs = []
            for k in range(NCK):
                lum = (px[k][0] + px[k][1] + px[k][2]) * (1.0 / 3.0)
                idx = jnp.clip(((lum - tmin_v) * scale_v).astype(jnp.int32),
                               0, 255)
                enh = plsc.load_gather(cdf_v, [idx])
                et = lum + a_v * (enh - lum)
                mask = lum > 1e-05
                g = et * (1.0 / jnp.where(mask, lum, 1.0))
                for ch in range(C):
                    orig = px[k][ch]
                    val = jnp.where(mask, g * orig, et)
                    val = jnp.where(hr_v, val, orig)
                    outs.append(jnp.clip(val, 0.0, 1.0))
            for k in range(NCK):
                for ch in range(C):
                    chan_v[ch, r, pl.ds(wo + k * L, L)] = outs[k * C + ch]
        return carry

    lax.fori_loop(0, 2, _do_half, 0)
    pltpu.sync_copy(chan_v, out_hbm.at[:, pl.ds(hs, T), pl.ds(ws, 2 * T)])


def kernel(img):
    return _clahe(img, jnp.asarray(_ALPHAS))


# speedup vs baseline: 124.4876x; 1.0271x over previous
import functools

import numpy as np
import jax
import jax.numpy as jnp
from jax import lax
from jax.experimental import pallas as pl
from jax.experimental.pallas import tpu as pltpu
from jax.experimental.pallas import tpu_sc as plsc

C, H, W = 3, 512, 512
GRID = 8
T = 64
NTILES = GRID * GRID
L = 16
NBINS = 256
NCH = NBINS // L
NCK = T // L
CL = 64.0

_rng = np.random.RandomState(0)
_ALPHAS = np.array([_rng.uniform(0.5, 1.0) for _ in range(NTILES)],
                   dtype=np.float32)

_mesh = plsc.VectorSubcoreMesh(core_axis_name="c", subcore_axis_name="s")


@functools.partial(
    pl.kernel,
    mesh=_mesh,
    out_type=jax.ShapeDtypeStruct((C, H, W), jnp.float32),
    compiler_params=pltpu.CompilerParams(use_tc_tiling_on_sc=True,
                                         needs_layout_passes=False),
    scratch_types=[
        pltpu.VMEM((C, T, 2 * T), jnp.float32),
        pltpu.VMEM((L * NBINS,), jnp.float32),
        pltpu.VMEM((NBINS,), jnp.float32),
        pltpu.VMEM((NTILES,), jnp.float32),
    ],
)
def _clahe(img_hbm, alphas_hbm, out_hbm, chan_v, hist_v, cdf_v, alpha_v):
    wid = lax.axis_index("s") * 2 + lax.axis_index("c")
    pltpu.sync_copy(alphas_hbm, alpha_v)

    lanes256 = lax.iota(jnp.int32, L) * NBINS
    ones = jnp.ones((L,), jnp.float32)
    zeros = jnp.zeros((L,), jnp.float32)

    def _zero(i, carry):
        for j in range(NCH):
            hist_v[pl.ds(i * NBINS + j * L, L)] = zeros
        return carry
    lax.fori_loop(0, L, _zero, 0)

    iB = wid // 4
    jB = wid % 4
    hs = iB * T
    ws = jB * (2 * T)
    pltpu.sync_copy(img_hbm.at[:, pl.ds(hs, T), pl.ds(ws, 2 * T)], chan_v)

    def _do_half(half, carry):
        t = iB * GRID + jB * 2 + half
        wo = half * T

        @plsc.parallel_loop(
            0, T, 1, unroll=2,
            carry=(jnp.full((L,), jnp.inf, jnp.float32),
                   jnp.full((L,), -jnp.inf, jnp.float32)))
        def _pass_a(r, mm):
            vmin, vmax = mm
            px = [[chan_v[ch, r, pl.ds(wo + k * L, L)] for ch in range(C)]
                  for k in range(NCK)]
            idxs = []
            for k in range(NCK):
                lum = (px[k][0] + px[k][1] + px[k][2]) * (1.0 / 3.0)
                vmin = jnp.minimum(vmin, lum)
                vmax = jnp.maximum(vmax, lum)
                bins = jnp.minimum((lum * 256.0).astype(jnp.int32), 255)
                idxs.append(lanes256 + bins)
            for k in range(NCK):
                plsc.addupdate_scatter(hist_v, [idxs[k]], ones)
            return vmin, vmax
        vmin, vmax = _pass_a
        tmin_v = jnp.broadcast_to(jnp.min(vmin), (L,))
        tmax_v = jnp.broadcast_to(jnp.max(vmax), (L,))

        @plsc.parallel_loop(0, L, 1, unroll=2, carry=(zeros,) * NCH)
        def _sum_rows(i, accs):
            loads = [hist_v[pl.ds(i * NBINS + j * L, L)] for j in range(NCH)]
            for j in range(NCH):
                hist_v[pl.ds(i * NBINS + j * L, L)] = zeros
            return tuple(accs[j] + loads[j] for j in range(NCH))
        accs = _sum_rows

        @pl.when(jnp.max(vmax) > jnp.min(vmin))
        def _enhance():
            exc = zeros
            clipped = []
            for j in range(NCH):
                hj = accs[j]
                exc = exc + jnp.maximum(hj - CL, 0.0)
                clipped.append(jnp.minimum(hj, CL))
            add_v = jnp.broadcast_to(jnp.sum(exc), (L,)) * (1.0 / 256.0)

            run = zeros
            for j in range(NCH):
                sl = pl.ds(j * L, L)
                cdf_v[sl] = plsc.cumsum(clipped[j] + add_v) + run
                run = plsc.load_gather(cdf_v, [jnp.full((L,), j * L + L - 1,
                                                        jnp.int32)])
            inv_total = 1.0 / run
            for j in range(NCH):
                sl = pl.ds(j * L, L)
                cdf_v[sl] = cdf_v[sl] * inv_total

            a_v = plsc.load_gather(alpha_v, [jnp.broadcast_to(t, (L,))])
            scale_v = 255.0 / (tmax_v - tmin_v)

            @plsc.parallel_loop(0, T, 1, unroll=2)
            def _pass_b(r):
                px = [[chan_v[ch, r, pl.ds(wo + k * L, L)]
                       for ch in range(C)] for k in range(NCK)]
                outs = []
                for k in range(NCK):
                    lum = (px[k][0] + px[k][1] + px[k][2]) * (1.0 / 3.0)
                    idx = jnp.minimum(
                        ((lum - tmin_v) * scale_v).astype(jnp.int32), 255)
                    enh = plsc.load_gather(cdf_v, [idx])
                    et = lum + a_v * (enh - lum)
                    mask = lum > 1e-05
                    g = et * (1.0 / jnp.where(mask, lum, 1.0))
                    for ch in range(C):
                        val = jnp.where(mask, g * px[k][ch], et)
                        outs.append(jnp.minimum(val, 1.0))
                for k in range(NCK):
                    for ch in range(C):
                        chan_v[ch, r, pl.ds(wo + k * L, L)] = outs[k * C + ch]
        return carry

    lax.fori_loop(0, 2, _do_half, 0)
    pltpu.sync_copy(chan_v, out_hbm.at[:, pl.ds(hs, T), pl.ds(ws, 2 * T)])


def kernel(img):
    return _clahe(img, jnp.asarray(_ALPHAS))
